# Initial kernel scaffold; baseline (speedup 1.0000x reference)
import functools

import jax
import jax.numpy as jnp
from jax import lax
from jax.experimental import pallas as pl
from jax.experimental.pallas import tpu as pltpu
from jax.experimental.pallas import tpu_sc as plsc

N = 10000
E = 320000
D = 128
H = 32
B = 128

NTILES = 32
CHUNK = 128
EP = 323584
ROWS_E = EP // CHUNK
RPT = ROWS_E // NTILES
N2 = 10016
NPT = N2 // NTILES
SEG_ROWS = N2 // 16

_mesh = plsc.VectorSubcoreMesh(core_axis_name="c", subcore_axis_name="s")


@functools.partial(
    pl.kernel,
    out_type=jax.ShapeDtypeStruct((2, N2, 16), jnp.float32),
    mesh=_mesh,
    scratch_types=[
        pltpu.VMEM((RPT, CHUNK), jnp.int32),
        pltpu.VMEM((CHUNK, 16), jnp.float32),
        pltpu.VMEM_SHARED((N2, 16), jnp.float32),
    ],
)
def _deg(dst2, ones16, zeros16, out, idx_d, ones_v, acc):
  core = lax.axis_index("c")
  sub = lax.axis_index("s")
  wid = core * 16 + sub
  pltpu.sync_copy(zeros16.at[pl.ds(sub * SEG_ROWS, SEG_ROWS)],
                  acc.at[pl.ds(sub * SEG_ROWS, SEG_ROWS)])
  pltpu.sync_copy(ones16, ones_v)
  pltpu.sync_copy(dst2.at[pl.ds(wid * RPT, RPT)], idx_d)
  plsc.subcore_barrier()

  def body(c, carry):
    pltpu.sync_copy(ones_v, acc.at[idx_d.at[c]], add=True)
    return carry

  lax.fori_loop(0, RPT, body, 0)
  plsc.subcore_barrier()
  pltpu.sync_copy(acc.at[pl.ds(sub * SEG_ROWS, SEG_ROWS)],
                  out.at[core, pl.ds(sub * SEG_ROWS, SEG_ROWS)])


@functools.partial(
    pl.kernel,
    out_type=jax.ShapeDtypeStruct((2, N2, H), jnp.float32),
    mesh=_mesh,
    scratch_types=[
        pltpu.VMEM((RPT, CHUNK), jnp.int32),
        pltpu.VMEM((RPT, CHUNK), jnp.int32),
        pltpu.VMEM((CHUNK, H), jnp.float32),
        pltpu.VMEM_SHARED((N2, H), jnp.float32),
        pltpu.SemaphoreType.DMA,
    ],
)
def _conv(hs, src2, dst2, zeros32, out, idx_s, idx_d, rows, acc, sem):
  core = lax.axis_index("c")
  sub = lax.axis_index("s")
  wid = core * 16 + sub
  pltpu.sync_copy(zeros32.at[pl.ds(sub * SEG_ROWS, SEG_ROWS)],
                  acc.at[pl.ds(sub * SEG_ROWS, SEG_ROWS)])
  pltpu.sync_copy(src2.at[pl.ds(wid * RPT, RPT)], idx_s)
  pltpu.sync_copy(dst2.at[pl.ds(wid * RPT, RPT)], idx_d)
  plsc.subcore_barrier()

  def body(c, carry):
    pltpu.async_copy(hs.at[idx_s.at[c]], rows, sem).wait()
    pltpu.sync_copy(rows, acc.at[idx_d.at[c]], add=True)
    return carry

  lax.fori_loop(0, RPT, body, 0)
  plsc.subcore_barrier()
  pltpu.sync_copy(acc.at[pl.ds(sub * SEG_ROWS, SEG_ROWS)],
                  out.at[core, pl.ds(sub * SEG_ROWS, SEG_ROWS)])


_ACC = 136 * H


@functools.partial(
    pl.kernel,
    out_type=[
        jax.ShapeDtypeStruct((NTILES, B * H), jnp.float32),
        jax.ShapeDtypeStruct((NTILES, B * H), jnp.float32),
    ],
    mesh=_mesh,
    scratch_types=[
        pltpu.VMEM((NPT * H,), jnp.float32),
        pltpu.VMEM((NPT,), jnp.int32),
        pltpu.VMEM((_ACC,), jnp.float32),
        pltpu.VMEM((_ACC,), jnp.float32),
    ],
)
def _pool(rows2, bids2, outs, outm, rows, bids, accs, accm):
  wid = lax.axis_index("c") * 16 + lax.axis_index("s")
  pltpu.sync_copy(rows2.at[wid], rows)
  pltpu.sync_copy(bids2.at[wid], bids)

  zero = jnp.zeros((16,), jnp.float32)
  ninf = jnp.full((16,), -jnp.inf, jnp.float32)

  def init(i, carry):
    accs[pl.ds(i * 16, 16)] = zero
    accm[pl.ds(i * 16, 16)] = ninf
    return carry

  lax.fori_loop(0, _ACC // 16, init, 0)

  def body(c, carry):
    b = bids[c]
    for h in (0, 16):
      r = rows[pl.ds(c * H + h, 16)]
      o = b * H + h
      accs[pl.ds(o, 16)] = accs[pl.ds(o, 16)] + r
      accm[pl.ds(o, 16)] = jnp.maximum(accm[pl.ds(o, 16)], r)
    return carry

  lax.fori_loop(0, NPT, body, 0)
  pltpu.sync_copy(accs.at[pl.ds(0, B * H)], outs.at[wid])
  pltpu.sync_copy(accm.at[pl.ds(0, B * H)], outm.at[wid])


_BN = 1000


def _prep_body(p0, p1, x, w, out_dis, out_hs):
  deg = p0[:, 0:1] + p1[:, 0:1] + 1.0
  dis = lax.rsqrt(deg)
  out_dis[...] = dis
  out_hs[...] = dis * jnp.dot(x[...], w[...],
                              preferred_element_type=jnp.float32)


def _tc_prep(p0, p1, x, w):
  return pl.pallas_call(
      _prep_body,
      grid=(N // _BN,),
      in_specs=[
          pl.BlockSpec((_BN, 16), lambda i: (i, 0)),
          pl.BlockSpec((_BN, 16), lambda i: (i, 0)),
          pl.BlockSpec((_BN, D), lambda i: (i, 0)),
          pl.BlockSpec((D, H), lambda i: (0, 0)),
      ],
      out_specs=[
          pl.BlockSpec((_BN, 1), lambda i: (i, 0)),
          pl.BlockSpec((_BN, H), lambda i: (i, 0)),
      ],
      out_shape=[
          jax.ShapeDtypeStruct((N, 1), jnp.float32),
          jax.ShapeDtypeStruct((N, H), jnp.float32),
      ],
  )(p0, p1, x, w)


def _mid_body(p0, p1, hs, dis, b, w, out):
  conv = dis[...] * (p0[...] + p1[...] + hs[...]) + b[...]
  xn = jnp.maximum(conv, 0.0)
  out[...] = dis[...] * jnp.dot(xn, w[...],
                                preferred_element_type=jnp.float32)


def _tc_mid(p0, p1, hs, dis, b, w):
  return pl.pallas_call(
      _mid_body,
      grid=(N // _BN,),
      in_specs=[
          pl.BlockSpec((_BN, H), lambda i: (i, 0)),
          pl.BlockSpec((_BN, H), lambda i: (i, 0)),
          pl.BlockSpec((_BN, H), lambda i: (i, 0)),
          pl.BlockSpec((_BN, 1), lambda i: (i, 0)),
          pl.BlockSpec((1, H), lambda i: (0, 0)),
          pl.BlockSpec((H, H), lambda i: (0, 0)),
      ],
      out_specs=pl.BlockSpec((_BN, H), lambda i: (i, 0)),
      out_shape=jax.ShapeDtypeStruct((N, H), jnp.float32),
  )(p0, p1, hs, dis, b, w)


def _post_body(p0, p1, hs, dis, b, out):
  conv = dis[...] * (p0[...] + p1[...] + hs[...]) + b[...]
  out[...] = jnp.maximum(conv, 0.0)


def _tc_post(p0, p1, hs, dis, b):
  return pl.pallas_call(
      _post_body,
      grid=(N // _BN,),
      in_specs=[
          pl.BlockSpec((_BN, H), lambda i: (i, 0)),
          pl.BlockSpec((_BN, H), lambda i: (i, 0)),
          pl.BlockSpec((_BN, H), lambda i: (i, 0)),
          pl.BlockSpec((_BN, 1), lambda i: (i, 0)),
          pl.BlockSpec((1, H), lambda i: (0, 0)),
      ],
      out_specs=pl.BlockSpec((_BN, H), lambda i: (i, 0)),
      out_shape=jax.ShapeDtypeStruct((N, H), jnp.float32),
  )(p0, p1, hs, dis, b)


def _final_body(sums, maxs, batch, w, b, out):
  gsum = jnp.sum(sums[...], axis=0)
  gmax = jnp.max(maxs[...], axis=0)
  ids = lax.broadcasted_iota(jnp.int32, (N, B), 1)
  cnt = jnp.sum((batch[...] == ids).astype(jnp.float32), axis=0)
  gmean = gsum / jnp.maximum(cnt, 1.0)[:, None]
  pooled = jnp.concatenate([gmax, gmean], axis=1)
  out[...] = jnp.dot(pooled, w[...],
                     preferred_element_type=jnp.float32) + b[...]


def _tc_final(sums, maxs, batch, w, b):
  return pl.pallas_call(
      _final_body,
      in_specs=[
          pl.BlockSpec((NTILES, B, H), lambda: (0, 0, 0)),
          pl.BlockSpec((NTILES, B, H), lambda: (0, 0, 0)),
          pl.BlockSpec((N, 1), lambda: (0, 0)),
          pl.BlockSpec((2 * H, 1), lambda: (0, 0)),
          pl.BlockSpec((1, 1), lambda: (0, 0)),
      ],
      out_specs=pl.BlockSpec((B, 1), lambda: (0, 0)),
      out_shape=jax.ShapeDtypeStruct((B, 1), jnp.float32),
  )(sums, maxs, batch, w, b)


def kernel(x, edge_index, batch_index, W_in, b_in, W1, b1, W2, b2, W3, b3,
           W_out, b_out):
  pad = EP - E
  src = jnp.concatenate([edge_index[0], jnp.zeros((pad,), jnp.int32)])
  dst = jnp.concatenate([edge_index[1], jnp.full((pad,), N, jnp.int32)])
  src2 = src.reshape(ROWS_E, CHUNK)
  dst2 = dst.reshape(ROWS_E, CHUNK)
  ones16 = jnp.ones((CHUNK, 16), jnp.float32)
  zeros16 = jnp.zeros((N2, 16), jnp.float32)
  zeros32 = jnp.zeros((N2, H), jnp.float32)

  degp = _deg(dst2, ones16, zeros16)
  dis, hs = _tc_prep(degp[0, :N], degp[1, :N], x, W_in)

  for (bb, ww) in ((b_in, W1), (b1, W2), (b2, W3)):
    p = _conv(hs, src2, dst2, zeros32)
    hs = _tc_mid(p[0, :N], p[1, :N], hs, dis, bb.reshape(1, H), ww)
  p = _conv(hs, src2, dst2, zeros32)
  out4 = _tc_post(p[0, :N], p[1, :N], hs, dis, b3.reshape(1, H))

  rows2 = jnp.concatenate(
      [out4, jnp.zeros((N2 - N, H), jnp.float32)]).reshape(NTILES, NPT * H)
  bids2 = jnp.concatenate(
      [batch_index, jnp.full((N2 - N,), B, jnp.int32)]).reshape(NTILES, NPT)
  sums, maxs = _pool(rows2, bids2)

  return _tc_final(sums.reshape(NTILES, B, H), maxs.reshape(NTILES, B, H),
                   batch_index.reshape(N, 1), W_out, b_out.reshape(1, 1))


# speedup vs baseline: 6.4517x; 6.4517x over previous
import functools

import jax
import jax.numpy as jnp
from jax import lax
from jax.experimental import pallas as pl
from jax.experimental.pallas import tpu as pltpu
from jax.experimental.pallas import tpu_sc as plsc

N = 10000
E = 320000
D = 128
H = 32
B = 128

NTILES = 32
CHUNK = 128
EP = 327680
ROWS_E = EP // CHUNK
RPT = ROWS_E // NTILES
N2 = 10112
SEG_ROWS = N2 // 16
NP3 = 10240
NPT = NP3 // NTILES

_mesh = plsc.VectorSubcoreMesh(core_axis_name="c", subcore_axis_name="s")


@functools.partial(
    pl.kernel,
    out_type=jax.ShapeDtypeStruct((2, N2, D), jnp.float32),
    mesh=_mesh,
    scratch_types=[
        pltpu.VMEM((RPT, CHUNK), jnp.int32),
        pltpu.VMEM((CHUNK, D), jnp.float32),
        pltpu.VMEM_SHARED((N2, D), jnp.float32),
    ],
)
def _deg(dst2, zeros32, out, idx_d, ones_v, acc):
  core = lax.axis_index("c")
  sub = lax.axis_index("s")
  wid = core * 16 + sub
  pltpu.sync_copy(zeros32.at[pl.ds(sub * SEG_ROWS, SEG_ROWS)],
                  acc.at[pl.ds(sub * SEG_ROWS, SEG_ROWS)])
  pltpu.sync_copy(dst2.at[pl.ds(wid * RPT, RPT)], idx_d)
  one = jnp.ones((16,), jnp.float32)

  def initones(i, carry):
    for h in range(0, D, 16):
      ones_v[i, pl.ds(h, 16)] = one
    return carry

  lax.fori_loop(0, CHUNK, initones, 0)
  plsc.subcore_barrier()

  def body(c, carry):
    pltpu.sync_copy(ones_v, acc.at[idx_d.at[c]], add=True)
    return carry

  lax.fori_loop(0, RPT, body, 0)
  plsc.subcore_barrier()
  pltpu.sync_copy(acc.at[pl.ds(sub * SEG_ROWS, SEG_ROWS)],
                  out.at[core, pl.ds(sub * SEG_ROWS, SEG_ROWS)])


@functools.partial(
    pl.kernel,
    out_type=jax.ShapeDtypeStruct((2, N2, D), jnp.float32),
    mesh=_mesh,
    scratch_types=[
        pltpu.VMEM((RPT, CHUNK), jnp.int32),
        pltpu.VMEM((RPT, CHUNK), jnp.int32),
        pltpu.VMEM((CHUNK, D), jnp.float32),
        pltpu.VMEM_SHARED((N2, D), jnp.float32),
        pltpu.SemaphoreType.DMA,
    ],
)
def _conv_wide(hs, src2, dst2, zerosw, out, idx_s, idx_d, rows, acc, sem):
  core = lax.axis_index("c")
  sub = lax.axis_index("s")
  wid = core * 16 + sub
  pltpu.sync_copy(zerosw.at[pl.ds(sub * SEG_ROWS, SEG_ROWS)],
                  acc.at[pl.ds(sub * SEG_ROWS, SEG_ROWS)])
  pltpu.sync_copy(src2.at[pl.ds(wid * RPT, RPT)], idx_s)
  pltpu.sync_copy(dst2.at[pl.ds(wid * RPT, RPT)], idx_d)
  plsc.subcore_barrier()

  def body(c, carry):
    pltpu.async_copy(hs.at[idx_s.at[c]], rows, sem).wait()
    pltpu.sync_copy(rows, acc.at[idx_d.at[c]], add=True)
    return carry

  lax.fori_loop(0, RPT, body, 0)
  plsc.subcore_barrier()
  pltpu.sync_copy(acc.at[pl.ds(sub * SEG_ROWS, SEG_ROWS)],
                  out.at[core, pl.ds(sub * SEG_ROWS, SEG_ROWS)])


_ACC = 136 * H


@functools.partial(
    pl.kernel,
    out_type=[
        jax.ShapeDtypeStruct((NTILES * B * H,), jnp.float32),
        jax.ShapeDtypeStruct((NTILES * B * H,), jnp.float32),
    ],
    mesh=_mesh,
    scratch_types=[
        pltpu.VMEM((NPT * H,), jnp.float32),
        pltpu.VMEM((NPT,), jnp.int32),
        pltpu.VMEM((_ACC,), jnp.float32),
        pltpu.VMEM((_ACC,), jnp.float32),
    ],
)
def _pool(rows2, bids2, outs, outm, rows, bids, accs, accm):
  wid = lax.axis_index("c") * 16 + lax.axis_index("s")
  pltpu.sync_copy(rows2.at[pl.ds(wid * (NPT * H), NPT * H)], rows)
  pltpu.sync_copy(bids2.at[pl.ds(wid * NPT, NPT)], bids)

  zero = jnp.zeros((16,), jnp.float32)
  ninf = jnp.full((16,), -jnp.inf, jnp.float32)

  def init(i, carry):
    accs[pl.ds(i * 16, 16)] = zero
    accm[pl.ds(i * 16, 16)] = ninf
    return carry

  lax.fori_loop(0, _ACC // 16, init, 0)

  def body(g, carry):
    bvec = bids[pl.ds(g * 16, 16)]
    for l in range(16):
      b = bvec[l]
      c = g * 16 + l
      for h in (0, 16):
        r = rows[pl.ds(c * H + h, 16)]
        o = b * H + h
        accs[pl.ds(o, 16)] = accs[pl.ds(o, 16)] + r
        accm[pl.ds(o, 16)] = jnp.maximum(accm[pl.ds(o, 16)], r)
    return carry

  lax.fori_loop(0, NPT // 16, body, 0)
  pltpu.sync_copy(accs.at[pl.ds(0, B * H)],
                  outs.at[pl.ds(wid * (B * H), B * H)])
  pltpu.sync_copy(accm.at[pl.ds(0, B * H)],
                  outm.at[pl.ds(wid * (B * H), B * H)])


_BN = 632
_GN = N2 // _BN


def _prep_body(p0, p1, x, w, out_dis, out_hs):
  deg = p0[:, 0:1] + p1[:, 0:1] + 1.0
  dis = lax.rsqrt(deg)
  out_dis[...] = dis
  hs = dis * jnp.dot(x[...], w[...], preferred_element_type=jnp.float32)
  out_hs[...] = jnp.concatenate(
      [hs, jnp.zeros((_BN, D - H), jnp.float32)], axis=1)


def _tc_prep(p0, p1, x, w):
  return pl.pallas_call(
      _prep_body,
      grid=(_GN,),
      in_specs=[
          pl.BlockSpec((_BN, D), lambda i: (i, 0)),
          pl.BlockSpec((_BN, D), lambda i: (i, 0)),
          pl.BlockSpec((_BN, D), lambda i: (i, 0)),
          pl.BlockSpec((D, H), lambda i: (0, 0)),
      ],
      out_specs=[
          pl.BlockSpec((_BN, 1), lambda i: (i, 0)),
          pl.BlockSpec((_BN, D), lambda i: (i, 0)),
      ],
      out_shape=[
          jax.ShapeDtypeStruct((N2, 1), jnp.float32),
          jax.ShapeDtypeStruct((N2, D), jnp.float32),
      ],
  )(p0, p1, x, w)


def _mid_body(p0, p1, hs, dis, b, w, out):
  conv = dis[...] * (p0[:, :H] + p1[:, :H] + hs[:, :H]) + b[...]
  xn = jnp.maximum(conv, 0.0)
  hsn = dis[...] * jnp.dot(xn, w[...], preferred_element_type=jnp.float32)
  out[...] = jnp.concatenate(
      [hsn, jnp.zeros((_BN, D - H), jnp.float32)], axis=1)


def _tc_mid(p0, p1, hs, dis, b, w):
  return pl.pallas_call(
      _mid_body,
      grid=(_GN,),
      in_specs=[
          pl.BlockSpec((_BN, D), lambda i: (i, 0)),
          pl.BlockSpec((_BN, D), lambda i: (i, 0)),
          pl.BlockSpec((_BN, D), lambda i: (i, 0)),
          pl.BlockSpec((_BN, 1), lambda i: (i, 0)),
          pl.BlockSpec((1, H), lambda i: (0, 0)),
          pl.BlockSpec((H, H), lambda i: (0, 0)),
      ],
      out_specs=pl.BlockSpec((_BN, D), lambda i: (i, 0)),
      out_shape=jax.ShapeDtypeStruct((N2, D), jnp.float32),
  )(p0, p1, hs, dis, b, w)


def _post_body(p0, p1, hs, dis, b, out):
  conv = dis[...] * (p0[:, :H] + p1[:, :H] + hs[:, :H]) + b[...]
  out[...] = jnp.maximum(conv, 0.0)


def _tc_post(p0, p1, hs, dis, b):
  return pl.pallas_call(
      _post_body,
      grid=(_GN,),
      in_specs=[
          pl.BlockSpec((_BN, D), lambda i: (i, 0)),
          pl.BlockSpec((_BN, D), lambda i: (i, 0)),
          pl.BlockSpec((_BN, D), lambda i: (i, 0)),
          pl.BlockSpec((_BN, 1), lambda i: (i, 0)),
          pl.BlockSpec((1, H), lambda i: (0, 0)),
      ],
      out_specs=pl.BlockSpec((_BN, H), lambda i: (i, 0)),
      out_shape=jax.ShapeDtypeStruct((N2, H), jnp.float32),
  )(p0, p1, hs, dis, b)


def _final_body(sums, maxs, batch, w, b, out):
  gsum = jnp.sum(sums[...], axis=0)
  gmax = jnp.max(maxs[...], axis=0)
  ids = lax.broadcasted_iota(jnp.int32, (N, B), 1)
  cnt = jnp.sum((batch[...] == ids).astype(jnp.float32), axis=0)
  gmean = gsum / jnp.maximum(cnt, 1.0)[:, None]
  pooled = jnp.concatenate([gmax, gmean], axis=1)
  out[...] = jnp.dot(pooled, w[...],
                     preferred_element_type=jnp.float32) + b[...]


def _tc_final(sums, maxs, batch, w, b):
  return pl.pallas_call(
      _final_body,
      in_specs=[
          pl.BlockSpec((NTILES, B, H), lambda: (0, 0, 0)),
          pl.BlockSpec((NTILES, B, H), lambda: (0, 0, 0)),
          pl.BlockSpec((N, 1), lambda: (0, 0)),
          pl.BlockSpec((2 * H, 1), lambda: (0, 0)),
          pl.BlockSpec((1, 1), lambda: (0, 0)),
      ],
      out_specs=pl.BlockSpec((B, 1), lambda: (0, 0)),
      out_shape=jax.ShapeDtypeStruct((B, 1), jnp.float32),
  )(sums, maxs, batch, w, b)


def kernel(x, edge_index, batch_index, W_in, b_in, W1, b1, W2, b2, W3, b3,
           W_out, b_out):
  pad = EP - E
  src = jnp.concatenate([edge_index[0], jnp.zeros((pad,), jnp.int32)])
  dst = jnp.concatenate([edge_index[1], jnp.full((pad,), N, jnp.int32)])
  src2 = src.reshape(ROWS_E, CHUNK)
  dst2 = dst.reshape(ROWS_E, CHUNK)
  zerosw = jnp.zeros((N2, D), jnp.float32)
  x2 = jnp.concatenate([x, jnp.zeros((N2 - N, D), jnp.float32)])

  degp = _deg(dst2, zerosw)
  dis, hs = _tc_prep(degp[0], degp[1], x2, W_in)

  for (bb, ww) in ((b_in, W1), (b1, W2), (b2, W3)):
    p = _conv_wide(hs, src2, dst2, zerosw)
    hs = _tc_mid(p[0], p[1], hs, dis, bb.reshape(1, H), ww)
  p = _conv_wide(hs, src2, dst2, zerosw)
  out4 = _tc_post(p[0], p[1], hs, dis, b3.reshape(1, H))

  rows2 = jnp.concatenate(
      [out4[:N], jnp.zeros((NP3 - N, H), jnp.float32)]).reshape(NP3 * H)
  bids2 = jnp.concatenate(
      [batch_index, jnp.full((NP3 - N,), B, jnp.int32)])
  sums, maxs = _pool(rows2, bids2)

  return _tc_final(sums.reshape(NTILES, B, H), maxs.reshape(NTILES, B, H),
                   batch_index.reshape(N, 1), W_out, b_out.reshape(1, 1))
